# Initial kernel scaffold; baseline (speedup 1.0000x reference)
import jax
import jax.numpy as jnp
from jax.experimental import pallas as pl


def kernel(x, edge_index, batch, params):
    raise NotImplementedError("write your pallas kernel here")


# speedup vs baseline: 6.7209x; 6.7209x over previous
import functools

import jax
import jax.numpy as jnp
from jax import lax
from jax.experimental import pallas as pl
from jax.experimental.pallas import tpu as pltpu
from jax.experimental.pallas import tpu_sc as plsc

_N = 10000
_E = 320000
_H = 128
_G = 64

_NC = 2
_NS = 16
_NW = _NC * _NS
_EPW = _E // _NW
_CH = 80
_NCHUNK = _EPW // _CH
_RCH = 80
_NRCH = _N // _RCH
_RPT = -(-_NRCH // _NS)
_ZR = 16
_NZCH = _N // _ZR
_ZPT = -(-_NZCH // _NS)


def _sc_agg(x, src3, dst3):
    mesh = plsc.VectorSubcoreMesh(core_axis_name="c", subcore_axis_name="s")

    @functools.partial(
        pl.kernel,
        out_type=jax.ShapeDtypeStruct((_NC, _N, _H), jnp.float32),
        mesh=mesh,
        scratch_types=[
            pltpu.VMEM((_NCHUNK, _CH), jnp.int32),
            pltpu.VMEM((_NCHUNK, _CH), jnp.int32),
            pltpu.VMEM((_CH, _H), jnp.float32),
            pltpu.VMEM((_ZR, _H), jnp.float32),
            pltpu.VMEM_SHARED((_N, _H), jnp.float32),
            pltpu.SemaphoreType.DMA,
        ],
    )
    def k(x_hbm, src_hbm, dst_hbm, out_hbm, sidx, didx, rows, zbuf, agg_sh, sem):
        c = lax.axis_index("c")
        s = lax.axis_index("s")
        wid = s * _NC + c

        pltpu.sync_copy(src_hbm.at[wid], sidx)
        pltpu.sync_copy(dst_hbm.at[wid], didx)

        @pl.loop(0, _ZR)
        def _(r):
            for j in range(_H // 16):
                zbuf[r, pl.ds(j * 16, 16)] = jnp.zeros((16,), jnp.float32)

        @pl.loop(0, _ZPT)
        def _(t):
            cid = t * _NS + s

            @pl.when(cid < _NZCH)
            def _():
                off = pl.multiple_of(cid * _ZR, 8)
                pltpu.sync_copy(zbuf, agg_sh.at[pl.ds(off, _ZR)])

        plsc.subcore_barrier()

        @pl.loop(0, _NCHUNK)
        def _(i):
            pltpu.async_copy(x_hbm.at[sidx.at[i]], rows, sem).wait()
            pltpu.sync_copy(rows, agg_sh.at[didx.at[i]], add=True)

        plsc.subcore_barrier()

        @pl.loop(0, _RPT)
        def _(t):
            cid = t * _NS + s

            @pl.when(cid < _NRCH)
            def _():
                off = pl.multiple_of(cid * _RCH, 8)
                pltpu.sync_copy(agg_sh.at[pl.ds(off, _RCH)],
                                out_hbm.at[c].at[pl.ds(off, _RCH)])

    return k(x, src3, dst3)


def _tc_layer(x, aggp, Wa, ba, Wb, bb, gamma, beta):
    def body(x_ref, a_ref, wa_ref, ba_ref, wb_ref, bb_ref, g_ref, be_ref, o_ref):
        h = x_ref[...] + a_ref[0] + a_ref[1]
        z = jnp.dot(h, wa_ref[...], preferred_element_type=jnp.float32)
        z = jnp.maximum(z + ba_ref[...], 0.0)
        z = jnp.dot(z, wb_ref[...], preferred_element_type=jnp.float32)
        z = jnp.maximum(z + bb_ref[...], 0.0)
        m = jnp.mean(z, axis=0, keepdims=True)
        d = z - m
        v = jnp.mean(d * d, axis=0, keepdims=True)
        o_ref[...] = d / jnp.sqrt(v + 1e-5) * g_ref[...] + be_ref[...]

    return pl.pallas_call(
        body, out_shape=jax.ShapeDtypeStruct((_N, _H), jnp.float32),
    )(x, aggp, Wa, ba.reshape(1, _H), Wb, bb.reshape(1, _H),
      gamma.reshape(1, _H), beta.reshape(1, _H))


def _tc_pool(x1, x2, x3, batch2d, W1, W2, W3, b):
    def body(x1_ref, x2_ref, x3_ref, bt_ref, w1_ref, w2_ref, w3_ref, b_ref,
             o_ref):
        gid = lax.broadcasted_iota(jnp.int32, (_G, _N), 0)
        onehot = (gid == bt_ref[...]).astype(jnp.float32)
        counts = jnp.sum(onehot, axis=1, keepdims=True)
        pt = onehot / jnp.maximum(counts, 1.0)
        acc = jnp.dot(jnp.dot(pt, x1_ref[...], preferred_element_type=jnp.float32, precision=lax.Precision.HIGHEST),
                      w1_ref[...], preferred_element_type=jnp.float32)
        acc += jnp.dot(jnp.dot(pt, x2_ref[...], preferred_element_type=jnp.float32, precision=lax.Precision.HIGHEST),
                       w2_ref[...], preferred_element_type=jnp.float32)
        acc += jnp.dot(jnp.dot(pt, x3_ref[...], preferred_element_type=jnp.float32, precision=lax.Precision.HIGHEST),
                       w3_ref[...], preferred_element_type=jnp.float32)
        o_ref[...] = acc + b_ref[...]

    return pl.pallas_call(
        body, out_shape=jax.ShapeDtypeStruct((_G, _H), jnp.float32),
    )(x1, x2, x3, batch2d, W1, W2, W3, b)


def kernel(x, edge_index, batch, params):
    src3 = edge_index[0].reshape(_NW, _NCHUNK, _CH)
    dst3 = edge_index[1].reshape(_NW, _NCHUNK, _CH)
    h = x
    feats = []
    for l in range(3):
        aggp = _sc_agg(h, src3, dst3)
        h = _tc_layer(h, aggp,
                      params['l%d_Wa' % l], params['l%d_ba' % l],
                      params['l%d_Wb' % l], params['l%d_bb' % l],
                      params['l%d_gamma' % l], params['l%d_beta' % l])
        feats.append(h)
    W = params['lin_W']
    return _tc_pool(feats[0], feats[1], feats[2], batch.reshape(1, _N),
                    W[0:_H], W[_H:2 * _H], W[2 * _H:3 * _H],
                    params['lin_b'].reshape(1, _H))


# speedup vs baseline: 10.9667x; 1.6317x over previous
import functools

import jax
import jax.numpy as jnp
from jax import lax
from jax.experimental import pallas as pl
from jax.experimental.pallas import tpu as pltpu
from jax.experimental.pallas import tpu_sc as plsc

_N = 10000
_E = 320000
_H = 128
_G = 64

_NC = 2
_NS = 16
_NW = _NC * _NS
_EPW = _E // _NW
_CH = 80
_NCHUNK = _EPW // _CH
_RCH = 80
_NRCH = _N // _RCH
_RPT = -(-_NRCH // _NS)
_ZR = 16
_NZCH = _N // _ZR
_ZPT = -(-_NZCH // _NS)
_RING = 3


def _sc_agg(x, src3, dst3):
    mesh = plsc.VectorSubcoreMesh(core_axis_name="c", subcore_axis_name="s")

    @functools.partial(
        pl.kernel,
        out_type=jax.ShapeDtypeStruct((_NC, _N, _H), jnp.float32),
        mesh=mesh,
        scratch_types=[
            pltpu.VMEM((_CH, _H), jnp.float32),
            pltpu.VMEM((_CH, _H), jnp.float32),
            pltpu.VMEM((_CH, _H), jnp.float32),
            pltpu.VMEM((_RING, 2, _CH), jnp.int32),
            pltpu.VMEM((16, _H), jnp.float32),
            pltpu.VMEM_SHARED((_N, _H), jnp.float32),
            pltpu.SemaphoreType.DMA,
            pltpu.SemaphoreType.DMA,
            pltpu.SemaphoreType.DMA,
            pltpu.SemaphoreType.DMA,
            pltpu.SemaphoreType.DMA,
            pltpu.SemaphoreType.DMA,
            pltpu.SemaphoreType.DMA,
            pltpu.SemaphoreType.DMA,
            pltpu.SemaphoreType.DMA,
        ],
    )
    def k(x_hbm, src_hbm, dst_hbm, out_hbm,
          rows0, rows1, rows2, idxr, zbuf, agg_sh,
          sg0, sg1, sg2, ss0, ss1, ss2, sd0, sd1, sd2):
        c = lax.axis_index("c")
        s = lax.axis_index("s")
        wid = s * _NC + c
        rows = (rows0, rows1, rows2)
        semg = (sg0, sg1, sg2)
        semis = (ss0, ss1, ss2)
        semid = (sd0, sd1, sd2)

        def idx_start(j, t):
            pltpu.make_async_copy(
                src_hbm.at[wid].at[j], idxr.at[t, 0], semis[t]).start()
            pltpu.make_async_copy(
                dst_hbm.at[wid].at[j], idxr.at[t, 1], semid[t]).start()

        def gather_start(t):
            pltpu.make_async_copy(
                x_hbm.at[idxr.at[t, 0]], rows[t], semg[t]).start()

        def gather_wait(t):
            pltpu.make_async_copy(
                x_hbm.at[idxr.at[t, 0]], rows[t], semg[t]).wait()

        def idx_wait(t, which):
            sem = semis[t] if which == 0 else semid[t]
            pltpu.make_async_copy(
                src_hbm.at[wid].at[0], idxr.at[t, which], sem).wait()

        @pl.loop(0, 16)
        def _(r):
            for j in range(_H // 16):
                zbuf[r, pl.ds(j * 16, 16)] = jnp.zeros((16,), jnp.float32)

        @pl.loop(0, _ZPT)
        def _(t):
            cid = t * _NS + s

            @pl.when(cid < _NZCH)
            def _():
                off = pl.multiple_of(cid * _ZR, 8)
                pltpu.sync_copy(zbuf, agg_sh.at[pl.ds(off, _ZR)])

        plsc.subcore_barrier()

        for t in range(_RING):
            idx_start(t, t)
        for t in range(2):
            idx_wait(t, 0)
            gather_start(t)

        @pl.loop(0, -(-_NCHUNK // _RING))
        def _(m):
            for t in range(_RING):
                j = m * _RING + t

                @pl.when(j < _NCHUNK)
                def _():
                    gather_wait(t)
                    idx_wait(t, 1)
                    pltpu.sync_copy(rows[t], agg_sh.at[idxr.at[t, 1]],
                                    add=True)

                @pl.when(j + _RING < _NCHUNK)
                def _():
                    idx_start(j + _RING, t)

                t2 = (t + 2) % _RING

                @pl.when(j + 2 < _NCHUNK)
                def _():
                    idx_wait(t2, 0)
                    gather_start(t2)

        plsc.subcore_barrier()

        @pl.loop(0, _RPT)
        def _(t):
            cid = t * _NS + s

            @pl.when(cid < _NRCH)
            def _():
                off = pl.multiple_of(cid * _RCH, 8)
                pltpu.sync_copy(agg_sh.at[pl.ds(off, _RCH)],
                                out_hbm.at[c].at[pl.ds(off, _RCH)])

    return k(x, src3, dst3)


def _tc_layer(x, aggp, Wa, ba, Wb, bb, gamma, beta):
    def body(x_ref, a_ref, wa_ref, ba_ref, wb_ref, bb_ref, g_ref, be_ref, o_ref):
        h = x_ref[...] + a_ref[0] + a_ref[1]
        z = jnp.dot(h, wa_ref[...], preferred_element_type=jnp.float32)
        z = jnp.maximum(z + ba_ref[...], 0.0)
        z = jnp.dot(z, wb_ref[...], preferred_element_type=jnp.float32)
        z = jnp.maximum(z + bb_ref[...], 0.0)
        m = jnp.mean(z, axis=0, keepdims=True)
        d = z - m
        v = jnp.mean(d * d, axis=0, keepdims=True)
        o_ref[...] = d / jnp.sqrt(v + 1e-5) * g_ref[...] + be_ref[...]

    return pl.pallas_call(
        body, out_shape=jax.ShapeDtypeStruct((_N, _H), jnp.float32),
    )(x, aggp, Wa, ba.reshape(1, _H), Wb, bb.reshape(1, _H),
      gamma.reshape(1, _H), beta.reshape(1, _H))


def _tc_pool(x1, x2, x3, batch2d, W1, W2, W3, b):
    def body(x1_ref, x2_ref, x3_ref, bt_ref, w1_ref, w2_ref, w3_ref, b_ref,
             o_ref):
        gid = lax.broadcasted_iota(jnp.int32, (_G, _N), 0)
        onehot = (gid == bt_ref[...]).astype(jnp.float32)
        counts = jnp.sum(onehot, axis=1, keepdims=True)
        pt = onehot / jnp.maximum(counts, 1.0)
        acc = jnp.dot(jnp.dot(pt, x1_ref[...], preferred_element_type=jnp.float32, precision=lax.Precision.HIGHEST),
                      w1_ref[...], preferred_element_type=jnp.float32)
        acc += jnp.dot(jnp.dot(pt, x2_ref[...], preferred_element_type=jnp.float32, precision=lax.Precision.HIGHEST),
                       w2_ref[...], preferred_element_type=jnp.float32)
        acc += jnp.dot(jnp.dot(pt, x3_ref[...], preferred_element_type=jnp.float32, precision=lax.Precision.HIGHEST),
                       w3_ref[...], preferred_element_type=jnp.float32)
        o_ref[...] = acc + b_ref[...]

    return pl.pallas_call(
        body, out_shape=jax.ShapeDtypeStruct((_G, _H), jnp.float32),
    )(x1, x2, x3, batch2d, W1, W2, W3, b)


def kernel(x, edge_index, batch, params):
    src3 = edge_index[0].reshape(_NW, _NCHUNK, _CH)
    dst3 = edge_index[1].reshape(_NW, _NCHUNK, _CH)
    h = x
    feats = []
    for l in range(3):
        aggp = _sc_agg(h, src3, dst3)
        h = _tc_layer(h, aggp,
                      params['l%d_Wa' % l], params['l%d_ba' % l],
                      params['l%d_Wb' % l], params['l%d_bb' % l],
                      params['l%d_gamma' % l], params['l%d_beta' % l])
        feats.append(h)
    W = params['lin_W']
    return _tc_pool(feats[0], feats[1], feats[2], batch.reshape(1, _N),
                    W[0:_H], W[_H:2 * _H], W[2 * _H:3 * _H],
                    params['lin_b'].reshape(1, _H))


# speedup vs baseline: 13.2714x; 1.2101x over previous
import functools

import jax
import jax.numpy as jnp
from jax import lax
from jax.experimental import pallas as pl
from jax.experimental.pallas import tpu as pltpu
from jax.experimental.pallas import tpu_sc as plsc

_N = 10000
_E = 320000
_H = 128
_G = 64

_NC = 2
_NS = 16
_NW = _NC * _NS
_EPW = _E // _NW
_CH = 80
_NCHUNK = _EPW // _CH
_RCH = 80
_NRCH = _N // _RCH
_RPT = -(-_NRCH // _NS)
_ZR = 16
_NZCH = _N // _ZR
_ZPT = -(-_NZCH // _NS)
_RING = 4


def _sc_agg(x, src3, dst3):
    mesh = plsc.VectorSubcoreMesh(core_axis_name="c", subcore_axis_name="s")

    @functools.partial(
        pl.kernel,
        out_type=jax.ShapeDtypeStruct((_NC, _N, _H), jnp.float32),
        mesh=mesh,
        scratch_types=[
            pltpu.VMEM((_CH, _H), jnp.float32),
            pltpu.VMEM((_CH, _H), jnp.float32),
            pltpu.VMEM((_CH, _H), jnp.float32),
            pltpu.VMEM((_CH, _H), jnp.float32),
            pltpu.VMEM((_RING, 2, _CH), jnp.int32),
            pltpu.VMEM((16, _H), jnp.float32),
            pltpu.VMEM_SHARED((_N, _H), jnp.float32),
            pltpu.SemaphoreType.DMA,
            pltpu.SemaphoreType.DMA,
            pltpu.SemaphoreType.DMA,
            pltpu.SemaphoreType.DMA,
            pltpu.SemaphoreType.DMA,
            pltpu.SemaphoreType.DMA,
            pltpu.SemaphoreType.DMA,
            pltpu.SemaphoreType.DMA,
            pltpu.SemaphoreType.DMA,
            pltpu.SemaphoreType.DMA,
            pltpu.SemaphoreType.DMA,
            pltpu.SemaphoreType.DMA,
            pltpu.SemaphoreType.DMA,
        ],
    )
    def k(x_hbm, src_hbm, dst_hbm, out_hbm,
          rows0, rows1, rows2, rows3, idxr, zbuf, agg_sh,
          sg0, sg1, sg2, sg3, ss0, ss1, ss2, ss3, sd0, sd1, sd2, sd3, semz):
        c = lax.axis_index("c")
        s = lax.axis_index("s")
        wid = s * _NC + c
        rows = (rows0, rows1, rows2, rows3)
        semg = (sg0, sg1, sg2, sg3)
        semis = (ss0, ss1, ss2, ss3)
        semid = (sd0, sd1, sd2, sd3)

        def idx_start(j, t):
            pltpu.make_async_copy(
                src_hbm.at[wid].at[j], idxr.at[t, 0], semis[t]).start()
            pltpu.make_async_copy(
                dst_hbm.at[wid].at[j], idxr.at[t, 1], semid[t]).start()

        def gather_start(t):
            pltpu.make_async_copy(
                x_hbm.at[idxr.at[t, 0]], rows[t], semg[t]).start()

        def gather_wait(t):
            pltpu.make_async_copy(
                x_hbm.at[idxr.at[t, 0]], rows[t], semg[t]).wait()

        def idx_wait(t, which):
            sem = semis[t] if which == 0 else semid[t]
            pltpu.make_async_copy(
                src_hbm.at[wid].at[0], idxr.at[t, which], sem).wait()

        for t in range(_RING):
            idx_start(t, t)
        for t in range(_RING - 1):
            idx_wait(t, 0)
            gather_start(t)

        @pl.loop(0, 16)
        def _(r):
            for j in range(_H // 16):
                zbuf[r, pl.ds(j * 16, 16)] = jnp.zeros((16,), jnp.float32)

        @pl.loop(0, _ZPT)
        def _(t):
            cid = t * _NS + s

            @pl.when(cid < _NZCH)
            def _():
                off = pl.multiple_of(cid * _ZR, 8)
                pltpu.make_async_copy(
                    zbuf, agg_sh.at[pl.ds(off, _ZR)], semz).start()

        @pl.loop(0, _ZPT)
        def _(t):
            cid = t * _NS + s

            @pl.when(cid < _NZCH)
            def _():
                pltpu.make_async_copy(
                    zbuf, agg_sh.at[pl.ds(0, _ZR)], semz).wait()

        plsc.subcore_barrier()

        @pl.loop(0, -(-_NCHUNK // _RING))
        def _(m):
            for t in range(_RING):
                j = m * _RING + t

                @pl.when(j < _NCHUNK)
                def _():
                    gather_wait(t)
                    idx_wait(t, 1)
                    pltpu.sync_copy(rows[t], agg_sh.at[idxr.at[t, 1]],
                                    add=True)

                @pl.when(j + _RING < _NCHUNK)
                def _():
                    idx_start(j + _RING, t)

                t2 = (t + _RING - 1) % _RING

                @pl.when(j + _RING - 1 < _NCHUNK)
                def _():
                    idx_wait(t2, 0)
                    gather_start(t2)

        plsc.subcore_barrier()

        @pl.loop(0, _RPT)
        def _(t):
            cid = t * _NS + s

            @pl.when(cid < _NRCH)
            def _():
                off = pl.multiple_of(cid * _RCH, 8)
                pltpu.make_async_copy(
                    agg_sh.at[pl.ds(off, _RCH)],
                    out_hbm.at[c].at[pl.ds(off, _RCH)], semz).start()

        @pl.loop(0, _RPT)
        def _(t):
            cid = t * _NS + s

            @pl.when(cid < _NRCH)
            def _():
                pltpu.make_async_copy(
                    agg_sh.at[pl.ds(0, _RCH)],
                    out_hbm.at[c].at[pl.ds(0, _RCH)], semz).wait()

    return k(x, src3, dst3)


def _tc_layer(x, aggp, Wa, ba, Wb, bb, gamma, beta):
    def body(x_ref, a_ref, wa_ref, ba_ref, wb_ref, bb_ref, g_ref, be_ref, o_ref):
        h = x_ref[...] + a_ref[0] + a_ref[1]
        z = jnp.dot(h, wa_ref[...], preferred_element_type=jnp.float32)
        z = jnp.maximum(z + ba_ref[...], 0.0)
        z = jnp.dot(z, wb_ref[...], preferred_element_type=jnp.float32)
        z = jnp.maximum(z + bb_ref[...], 0.0)
        m = jnp.mean(z, axis=0, keepdims=True)
        d = z - m
        v = jnp.mean(d * d, axis=0, keepdims=True)
        o_ref[...] = d / jnp.sqrt(v + 1e-5) * g_ref[...] + be_ref[...]

    return pl.pallas_call(
        body, out_shape=jax.ShapeDtypeStruct((_N, _H), jnp.float32),
    )(x, aggp, Wa, ba.reshape(1, _H), Wb, bb.reshape(1, _H),
      gamma.reshape(1, _H), beta.reshape(1, _H))


def _tc_pool(x1, x2, x3, batch2d, W1, W2, W3, b):
    def body(x1_ref, x2_ref, x3_ref, bt_ref, w1_ref, w2_ref, w3_ref, b_ref,
             o_ref):
        gid = lax.broadcasted_iota(jnp.int32, (_G, _N), 0)
        onehot = (gid == bt_ref[...]).astype(jnp.float32)
        counts = jnp.sum(onehot, axis=1, keepdims=True)
        pt = onehot / jnp.maximum(counts, 1.0)
        acc = jnp.dot(jnp.dot(pt, x1_ref[...], preferred_element_type=jnp.float32, precision=lax.Precision.HIGHEST),
                      w1_ref[...], preferred_element_type=jnp.float32)
        acc += jnp.dot(jnp.dot(pt, x2_ref[...], preferred_element_type=jnp.float32, precision=lax.Precision.HIGHEST),
                       w2_ref[...], preferred_element_type=jnp.float32)
        acc += jnp.dot(jnp.dot(pt, x3_ref[...], preferred_element_type=jnp.float32, precision=lax.Precision.HIGHEST),
                       w3_ref[...], preferred_element_type=jnp.float32)
        o_ref[...] = acc + b_ref[...]

    return pl.pallas_call(
        body, out_shape=jax.ShapeDtypeStruct((_G, _H), jnp.float32),
    )(x1, x2, x3, batch2d, W1, W2, W3, b)


def kernel(x, edge_index, batch, params):
    src3 = edge_index[0].reshape(_NW, _NCHUNK, _CH)
    dst3 = edge_index[1].reshape(_NW, _NCHUNK, _CH)
    h = x
    feats = []
    for l in range(3):
        aggp = _sc_agg(h, src3, dst3)
        h = _tc_layer(h, aggp,
                      params['l%d_Wa' % l], params['l%d_ba' % l],
                      params['l%d_Wb' % l], params['l%d_bb' % l],
                      params['l%d_gamma' % l], params['l%d_beta' % l])
        feats.append(h)
    W = params['lin_W']
    return _tc_pool(feats[0], feats[1], feats[2], batch.reshape(1, _N),
                    W[0:_H], W[_H:2 * _H], W[2 * _H:3 * _H],
                    params['lin_b'].reshape(1, _H))
